# baseline (speedup 1.0000x reference)
import jax
import jax.numpy as jnp
from jax import lax
from jax.experimental import pallas as pl
from jax.experimental.pallas import tpu as pltpu
from jax.experimental.pallas import tpu_sc as plsc

B, C, H, W = 16, 3, 512, 512
NCH = B * C
PIX = H * W
NC, NS, L = 2, 16, 16
CH_PER_CORE = NCH // NC
CHUNK = PIX // NS
ROWS = CHUNK // W
GROUPS = 256 // L
NBINS_ALL = CH_PER_CORE * 256


def _bins_off(v, off):
    return (v * 255.0 + (0.5 + off * 256.0)).astype(jnp.int32)


def _sc_body(x_hbm, out_hbm, bins_hbm, in_a, in_b, out_a, out_b, bb_a, bb_b,
             hist_all, lut_all, hist_v, lut_v, pbuf_v, part_sh, lut_sh,
             sem_ia, sem_ib, sem_oa, sem_ob, sem_ba, sem_bb):
    c = lax.axis_index("c")
    s = lax.axis_index("s")
    base_ch = c * CH_PER_CORE
    iota = lax.iota(jnp.int32, L)
    ones = jnp.ones((L,), jnp.int32)
    in_bufs = (in_a, in_b)
    out_bufs = (out_a, out_b)
    in_sems = (sem_ia, sem_ib)
    out_sems = (sem_oa, sem_ob)
    b_bufs = (bb_a, bb_b)
    b_sems = (sem_ba, sem_bb)

    def chunk_slice(i):
        return (base_ch + i, pl.ds(s * ROWS, ROWS), slice(None))

    def bchunk_slice(i):
        return (base_ch + i, pl.ds(s * ROWS, ROWS), slice(None))

    in3_bufs = (in_a, in_b, out_a)
    in3_sems = (sem_ia, sem_ib, sem_oa)
    in_cps = [None, None, None]
    in_cps[0] = pltpu.async_copy(x_hbm.at[chunk_slice(0)], in3_bufs[0],
                                 in3_sems[0])
    in_cps[1] = pltpu.async_copy(x_hbm.at[chunk_slice(1)], in3_bufs[1],
                                 in3_sems[1])

    @plsc.parallel_loop(0, NBINS_ALL, step=L, unroll=8)
    def _(j):
        hist_all[pl.ds(j, L)] = jnp.zeros((L,), jnp.int32)

    b_cps = [None, None]
    for i in range(CH_PER_CORE):
        if i + 2 < CH_PER_CORE:
            in_cps[(i + 2) % 3] = pltpu.async_copy(
                x_hbm.at[chunk_slice(i + 2)], in3_bufs[(i + 2) % 3],
                in3_sems[(i + 2) % 3])
        in_cps[i % 3].wait()
        if b_cps[i % 2] is not None:
            b_cps[i % 2].wait()
        buf = in3_bufs[i % 3]
        bbuf = b_bufs[i % 2]

        @plsc.parallel_loop(0, CHUNK, step=4 * L, unroll=4)
        def _(j):
            r = lax.shift_right_logical(j, 9)
            q = j & (W - 1)
            b0 = _bins_off(buf[r, pl.ds(q, L)], i)
            b1 = _bins_off(buf[r, pl.ds(q + L, L)], i)
            b2 = _bins_off(buf[r, pl.ds(q + 2 * L, L)], i)
            b3 = _bins_off(buf[r, pl.ds(q + 3 * L, L)], i)
            plsc.addupdate_scatter(hist_all, [b0], ones)
            plsc.addupdate_scatter(hist_all, [b1], ones)
            plsc.addupdate_scatter(hist_all, [b2], ones)
            plsc.addupdate_scatter(hist_all, [b3], ones)
            base = jnp.int32(i * 256)
            p = ((b0 - base) | lax.shift_left(b1 - base, 8)
                 | lax.shift_left(b2 - base, 16)
                 | lax.shift_left(b3 - base, 24))
            bbuf[r, pl.ds(lax.shift_right_logical(q, 2), L)] = p

        b_cps[i % 2] = pltpu.async_copy(bbuf, bins_hbm.at[bchunk_slice(i)],
                                        b_sems[i % 2])
    b_cps[0].wait()
    b_cps[1].wait()
    b_cps[0] = pltpu.async_copy(bins_hbm.at[bchunk_slice(0)], b_bufs[0],
                                b_sems[0])
    b_cps[1] = pltpu.async_copy(bins_hbm.at[bchunk_slice(1)], b_bufs[1],
                                b_sems[1])
    pltpu.sync_copy(hist_all, part_sh.at[s])

    plsc.subcore_barrier()

    magic = jnp.full((L,), 0x7EF311C3, jnp.int32)

    def floordiv(n, d):
        df = d.astype(jnp.float32)
        r = plsc.bitcast(magic - plsc.bitcast(df, jnp.int32), jnp.float32)
        for _ in range(3):
            r = r * (jnp.float32(2.0) - df * r)
        q = (n.astype(jnp.float32) * r).astype(jnp.int32)
        for _ in range(2):
            rem = n - q * d
            q = (q + jnp.where(rem >= d, 1, 0)) - jnp.where(rem < 0, 1, 0)
        return q

    def make_lut(ci):
        pltpu.sync_copy(part_sh.at[:, pl.ds(ci * 256, 256)], pbuf_v)
        last_comb = jnp.full((L,), -1, jnp.int32)
        for g in range(GROUPS):
            acc = pbuf_v[0, pl.ds(g * L, L)]
            for t in range(1, NS):
                acc = acc + pbuf_v[t, pl.ds(g * L, L)]
            hist_v[pl.ds(g * L, L)] = acc
            idxv = iota + g * L
            comb = jnp.where(acc > 0, lax.shift_left(idxv, 19) + acc, -1)
            last_comb = jnp.maximum(last_comb, comb)
        last_val = jnp.broadcast_to(
            jnp.bitwise_and(jnp.max(last_comb), jnp.int32((1 << 19) - 1)),
            (L,))
        step_v = floordiv(jnp.int32(PIX) - last_val,
                          jnp.full((L,), 255, jnp.int32))
        half_v = lax.shift_right_logical(step_v, 1)
        denom_v = jnp.maximum(step_v, 1)
        is_id = step_v == 0
        inv255 = jnp.float32(1.0 / 255.0)

        lut_v[pl.ds(0, L)] = jnp.zeros((L,), jnp.float32)
        carry = jnp.zeros((L,), jnp.int32)
        for g in range(GROUPS):
            hv = hist_v[pl.ds(g * L, L)]
            cum = carry + plsc.cumsum(hv)
            carry = jnp.broadcast_to(jnp.max(cum), (L,))
            q = floordiv(cum + half_v, denom_v)
            qc = jnp.clip(q, 0, 255).astype(jnp.float32)
            pos = iota + (g * L + 1)
            val = jnp.where(is_id, pos.astype(jnp.float32), qc) * inv255
            plsc.store_scatter(lut_v, [pos], val, mask=pos < 256)
        pltpu.sync_copy(lut_v, lut_sh.at[pl.ds(ci * 256, 256)])

    make_lut(s)

    @pl.when(s < CH_PER_CORE - NS)
    def _():
        make_lut(s + NS)

    plsc.subcore_barrier()

    pltpu.sync_copy(lut_sh, lut_all)
    out_cps = [None, None]
    mask255 = jnp.full((L,), 255, jnp.int32)
    for i in range(CH_PER_CORE):
        if 1 <= i and i + 1 < CH_PER_CORE:
            b_cps[(i + 1) % 2] = pltpu.async_copy(
                bins_hbm.at[bchunk_slice(i + 1)], b_bufs[(i + 1) % 2],
                b_sems[(i + 1) % 2])
        b_cps[i % 2].wait()
        if out_cps[i % 2] is not None:
            out_cps[i % 2].wait()
        bbuf = b_bufs[i % 2]
        obuf = out_bufs[i % 2]
        base = jnp.int32(i * 256)

        @plsc.parallel_loop(0, CHUNK, step=4 * L, unroll=4)
        def _(j):
            r = lax.shift_right_logical(j, 9)
            q = j & (W - 1)
            p = bbuf[r, pl.ds(lax.shift_right_logical(q, 2), L)]
            b0 = (p & mask255) + base
            b1 = (lax.shift_right_logical(p, 8) & mask255) + base
            b2 = (lax.shift_right_logical(p, 16) & mask255) + base
            b3 = lax.shift_right_logical(p, 24) + base
            obuf[r, pl.ds(q, L)] = plsc.load_gather(lut_all, [b0])
            obuf[r, pl.ds(q + L, L)] = plsc.load_gather(lut_all, [b1])
            obuf[r, pl.ds(q + 2 * L, L)] = plsc.load_gather(lut_all, [b2])
            obuf[r, pl.ds(q + 3 * L, L)] = plsc.load_gather(lut_all, [b3])

        out_cps[i % 2] = pltpu.async_copy(obuf, out_hbm.at[chunk_slice(i)],
                                          out_sems[i % 2])
    out_cps[0].wait()
    out_cps[1].wait()


@jax.jit
def _equalize_sc(x_flat):
    mesh = plsc.VectorSubcoreMesh(core_axis_name="c", subcore_axis_name="s")
    kfn = pl.kernel(
        _sc_body,
        out_type=(jax.ShapeDtypeStruct((NCH, H, W), jnp.float32),
                  jax.ShapeDtypeStruct((NCH, H, W // 4), jnp.int32)),
        mesh=mesh,
        compiler_params=pltpu.CompilerParams(needs_layout_passes=False),
        scratch_types=[
            pltpu.VMEM((ROWS, W), jnp.float32),
            pltpu.VMEM((ROWS, W), jnp.float32),
            pltpu.VMEM((ROWS, W), jnp.float32),
            pltpu.VMEM((ROWS, W), jnp.float32),
            pltpu.VMEM((ROWS, W // 4), jnp.int32),
            pltpu.VMEM((ROWS, W // 4), jnp.int32),
            pltpu.VMEM((NBINS_ALL,), jnp.int32),
            pltpu.VMEM((NBINS_ALL,), jnp.float32),
            pltpu.VMEM((256,), jnp.int32),
            pltpu.VMEM((256,), jnp.float32),
            pltpu.VMEM((NS, 256), jnp.int32),
            pltpu.VMEM_SHARED((NS, NBINS_ALL), jnp.int32),
            pltpu.VMEM_SHARED((NBINS_ALL,), jnp.float32),
            pltpu.SemaphoreType.DMA,
            pltpu.SemaphoreType.DMA,
            pltpu.SemaphoreType.DMA,
            pltpu.SemaphoreType.DMA,
            pltpu.SemaphoreType.DMA,
            pltpu.SemaphoreType.DMA,
        ],
    )
    return kfn(x_flat)


def kernel(x):
    out, _ = _equalize_sc(x.reshape(NCH, H, W))
    return out.reshape(B, C, H, W)


# speedup vs baseline: 1.0078x; 1.0078x over previous
import jax
import jax.numpy as jnp
from jax import lax
from jax.experimental import pallas as pl
from jax.experimental.pallas import tpu as pltpu
from jax.experimental.pallas import tpu_sc as plsc

B, C, H, W = 16, 3, 512, 512
NCH = B * C
PIX = H * W
NC, NS, L = 2, 16, 16
CH_PER_CORE = NCH // NC
CHUNK = PIX // NS
ROWS = CHUNK // W
GROUPS = 256 // L
NBINS_ALL = CH_PER_CORE * 256


def _bins_off(v, off):
    return (v * 255.0 + (0.5 + off * 256.0)).astype(jnp.int32)


def _sc_body(x_hbm, out_hbm, bins_hbm, in_a, in_b, out_a, out_b, bb_a, bb_b,
             hist_all, lut_all, hist_v, lut_v, pbuf_v, part_sh, lut_sh,
             sem_ia, sem_ib, sem_oa, sem_ob, sem_ba, sem_bb):
    c = lax.axis_index("c")
    s = lax.axis_index("s")
    base_ch = c * CH_PER_CORE
    iota = lax.iota(jnp.int32, L)
    ones = jnp.ones((L,), jnp.int32)
    in_bufs = (in_a, in_b)
    out_bufs = (out_a, out_b)
    in_sems = (sem_ia, sem_ib)
    out_sems = (sem_oa, sem_ob)
    b_bufs = (bb_a, bb_b)
    b_sems = (sem_ba, sem_bb)

    def chunk_slice(i):
        return (base_ch + i, pl.ds(s * ROWS, ROWS), slice(None))

    def bchunk_slice(i):
        return (base_ch + i, pl.ds(s * ROWS, ROWS), slice(None))

    cp = pltpu.async_copy(x_hbm.at[chunk_slice(0)], in_bufs[0], in_sems[0])

    @plsc.parallel_loop(0, NBINS_ALL, step=L, unroll=8)
    def _(j):
        hist_all[pl.ds(j, L)] = jnp.zeros((L,), jnp.int32)

    b_cps = [None, None]
    for i in range(CH_PER_CORE):
        nxt = None
        if i + 1 < CH_PER_CORE:
            nxt = pltpu.async_copy(x_hbm.at[chunk_slice(i + 1)],
                                   in_bufs[(i + 1) % 2],
                                   in_sems[(i + 1) % 2])
        cp.wait()
        if b_cps[i % 2] is not None:
            b_cps[i % 2].wait()
        buf = in_bufs[i % 2]
        bbuf = b_bufs[i % 2]

        @plsc.parallel_loop(0, CHUNK, step=4 * L, unroll=4)
        def _(j):
            r = lax.shift_right_logical(j, 9)
            q = j & (W - 1)
            b0 = _bins_off(buf[r, pl.ds(q, L)], i)
            b1 = _bins_off(buf[r, pl.ds(q + L, L)], i)
            b2 = _bins_off(buf[r, pl.ds(q + 2 * L, L)], i)
            b3 = _bins_off(buf[r, pl.ds(q + 3 * L, L)], i)
            plsc.addupdate_scatter(hist_all, [b0], ones)
            plsc.addupdate_scatter(hist_all, [b1], ones)
            plsc.addupdate_scatter(hist_all, [b2], ones)
            plsc.addupdate_scatter(hist_all, [b3], ones)
            base = jnp.int32(i * 256)
            p = ((b0 - base) | lax.shift_left(b1 - base, 8)
                 | lax.shift_left(b2 - base, 16)
                 | lax.shift_left(b3 - base, 24))
            bbuf[r, pl.ds(lax.shift_right_logical(q, 2), L)] = p

        b_cps[i % 2] = pltpu.async_copy(bbuf, bins_hbm.at[bchunk_slice(i)],
                                        b_sems[i % 2])
        cp = nxt
    b_cps[0].wait()
    b_cps[1].wait()
    b_cps[0] = pltpu.async_copy(bins_hbm.at[bchunk_slice(0)], b_bufs[0],
                                b_sems[0])
    b_cps[1] = pltpu.async_copy(bins_hbm.at[bchunk_slice(1)], b_bufs[1],
                                b_sems[1])
    pltpu.sync_copy(hist_all, part_sh.at[s])

    plsc.subcore_barrier()

    magic = jnp.full((L,), 0x7EF311C3, jnp.int32)

    def floordiv(n, d):
        df = d.astype(jnp.float32)
        r = plsc.bitcast(magic - plsc.bitcast(df, jnp.int32), jnp.float32)
        for _ in range(3):
            r = r * (jnp.float32(2.0) - df * r)
        q = (n.astype(jnp.float32) * r).astype(jnp.int32)
        for _ in range(2):
            rem = n - q * d
            q = (q + jnp.where(rem >= d, 1, 0)) - jnp.where(rem < 0, 1, 0)
        return q

    def make_lut(ci):
        pltpu.sync_copy(part_sh.at[:, pl.ds(ci * 256, 256)], pbuf_v)
        last_comb = jnp.full((L,), -1, jnp.int32)
        for g in range(GROUPS):
            acc = pbuf_v[0, pl.ds(g * L, L)]
            for t in range(1, NS):
                acc = acc + pbuf_v[t, pl.ds(g * L, L)]
            hist_v[pl.ds(g * L, L)] = acc
            idxv = iota + g * L
            comb = jnp.where(acc > 0, lax.shift_left(idxv, 19) + acc, -1)
            last_comb = jnp.maximum(last_comb, comb)
        last_val = jnp.broadcast_to(
            jnp.bitwise_and(jnp.max(last_comb), jnp.int32((1 << 19) - 1)),
            (L,))
        step_v = floordiv(jnp.int32(PIX) - last_val,
                          jnp.full((L,), 255, jnp.int32))
        half_v = lax.shift_right_logical(step_v, 1)
        denom_v = jnp.maximum(step_v, 1)
        is_id = step_v == 0
        inv255 = jnp.float32(1.0 / 255.0)

        lut_v[pl.ds(0, L)] = jnp.zeros((L,), jnp.float32)
        carry = jnp.zeros((L,), jnp.int32)
        for g in range(GROUPS):
            hv = hist_v[pl.ds(g * L, L)]
            cum = carry + plsc.cumsum(hv)
            carry = jnp.broadcast_to(jnp.max(cum), (L,))
            q = floordiv(cum + half_v, denom_v)
            qc = jnp.clip(q, 0, 255).astype(jnp.float32)
            pos = iota + (g * L + 1)
            val = jnp.where(is_id, pos.astype(jnp.float32), qc) * inv255
            plsc.store_scatter(lut_v, [pos], val, mask=pos < 256)
        pltpu.sync_copy(lut_v, lut_sh.at[pl.ds(ci * 256, 256)])

    make_lut(s)

    @pl.when(s < CH_PER_CORE - NS)
    def _():
        make_lut(s + NS)

    plsc.subcore_barrier()

    pltpu.sync_copy(lut_sh, lut_all)
    out_cps = [None, None]
    mask255 = jnp.full((L,), 255, jnp.int32)
    for i in range(CH_PER_CORE):
        if 1 <= i and i + 1 < CH_PER_CORE:
            b_cps[(i + 1) % 2] = pltpu.async_copy(
                bins_hbm.at[bchunk_slice(i + 1)], b_bufs[(i + 1) % 2],
                b_sems[(i + 1) % 2])
        b_cps[i % 2].wait()
        if out_cps[i % 2] is not None:
            out_cps[i % 2].wait()
        bbuf = b_bufs[i % 2]
        obuf = out_bufs[i % 2]
        base = jnp.int32(i * 256)

        @plsc.parallel_loop(0, CHUNK, step=4 * L, unroll=4)
        def _(j):
            r = lax.shift_right_logical(j, 9)
            q = j & (W - 1)
            p = bbuf[r, pl.ds(lax.shift_right_logical(q, 2), L)]
            b0 = (p & mask255) + base
            b1 = (lax.shift_right_logical(p, 8) & mask255) + base
            b2 = (lax.shift_right_logical(p, 16) & mask255) + base
            b3 = lax.shift_right_logical(p, 24) + base
            obuf[r, pl.ds(q, L)] = plsc.load_gather(lut_all, [b0])
            obuf[r, pl.ds(q + L, L)] = plsc.load_gather(lut_all, [b1])
            obuf[r, pl.ds(q + 2 * L, L)] = plsc.load_gather(lut_all, [b2])
            obuf[r, pl.ds(q + 3 * L, L)] = plsc.load_gather(lut_all, [b3])

        out_cps[i % 2] = pltpu.async_copy(obuf, out_hbm.at[chunk_slice(i)],
                                          out_sems[i % 2])
        cp = nxt
    out_cps[0].wait()
    out_cps[1].wait()


@jax.jit
def _equalize_sc(x_flat):
    mesh = plsc.VectorSubcoreMesh(core_axis_name="c", subcore_axis_name="s")
    kfn = pl.kernel(
        _sc_body,
        out_type=(jax.ShapeDtypeStruct((NCH, H, W), jnp.float32),
                  jax.ShapeDtypeStruct((NCH, H, W // 4), jnp.int32)),
        mesh=mesh,
        compiler_params=pltpu.CompilerParams(needs_layout_passes=False),
        scratch_types=[
            pltpu.VMEM((ROWS, W), jnp.float32),
            pltpu.VMEM((ROWS, W), jnp.float32),
            pltpu.VMEM((ROWS, W), jnp.float32),
            pltpu.VMEM((ROWS, W), jnp.float32),
            pltpu.VMEM((ROWS, W // 4), jnp.int32),
            pltpu.VMEM((ROWS, W // 4), jnp.int32),
            pltpu.VMEM((NBINS_ALL,), jnp.int32),
            pltpu.VMEM((NBINS_ALL,), jnp.float32),
            pltpu.VMEM((256,), jnp.int32),
            pltpu.VMEM((256,), jnp.float32),
            pltpu.VMEM((NS, 256), jnp.int32),
            pltpu.VMEM_SHARED((NS, NBINS_ALL), jnp.int32),
            pltpu.VMEM_SHARED((NBINS_ALL,), jnp.float32),
            pltpu.SemaphoreType.DMA,
            pltpu.SemaphoreType.DMA,
            pltpu.SemaphoreType.DMA,
            pltpu.SemaphoreType.DMA,
            pltpu.SemaphoreType.DMA,
            pltpu.SemaphoreType.DMA,
        ],
    )
    return kfn(x_flat)


def kernel(x):
    out, _ = _equalize_sc(x.reshape(NCH, H, W))
    return out.reshape(B, C, H, W)
